# baseline (speedup 1.0000x reference)
import jax
import jax.numpy as jnp
from jax import lax
from jax.experimental import pallas as pl
from jax.experimental.pallas import tpu as pltpu
from jax.experimental.pallas import tpu_sc as plsc

GLOBAL_MEAN = 3.5
D = 32
RW = 128
NC = 2
NS = 16
NW = NC * NS
L = 16
CH = 128


def _mf_body(sid_hbm, pid_hbm, p_hbm, q_hbm, sb_hbm, pb_hbm, out_hbm,
             sid_v, pid_v, is_v, ip_v, pbuf, qbuf, bs_v, bp_v, out_v,
             sem0, sem1, semb):
    b_per_w = sid_v.shape[0]
    nch = b_per_w // CH
    wid = lax.axis_index("s") * NC + lax.axis_index("c")
    base = wid * b_per_w
    sems = (sem0, sem1)

    pltpu.sync_copy(sid_hbm.at[wid], sid_v)
    pltpu.sync_copy(pid_hbm.at[wid], pid_v)

    bias_copies = []
    for k in range(nch):
        sl = pl.ds(k * CH, CH)
        bias_copies.append(pltpu.async_copy(sb_hbm.at[sid_v.at[sl]],
                                            bs_v.at[sl], semb))
        bias_copies.append(pltpu.async_copy(pb_hbm.at[pid_v.at[sl]],
                                            bp_v.at[sl], semb))

    lane = lax.iota(jnp.int32, L)
    for g in range(b_per_w // L):
        sl = pl.ds(g * L, L)
        is_v[sl] = lax.shift_right_logical(sid_v[sl], 2)
        ip_v[sl] = lax.shift_right_logical(pid_v[sl], 2)

    def fire(c):
        sl = pl.ds(c * CH, CH)
        buf = c % 2
        s = sems[buf]
        return (pltpu.async_copy(p_hbm.at[is_v.at[sl]], pbuf.at[buf], s),
                pltpu.async_copy(q_hbm.at[ip_v.at[sl]], qbuf.at[buf], s))

    inflight = fire(0)
    for c in range(nch):
        nxt = fire(c + 1) if c + 1 < nch else None
        for h in inflight:
            h.wait()
        if c == 0:
            for h in bias_copies:
                h.wait()
        buf = c % 2
        for g in range(CH // L):
            e0 = c * CH + g * L
            rows = g * L + lane
            sv = sid_v[pl.ds(e0, L)]
            pv = pid_v[pl.ds(e0, L)]
            offs = lax.shift_left(jnp.bitwise_and(sv, 3), 5)
            offp = lax.shift_left(jnp.bitwise_and(pv, 3), 5)
            acc = bs_v[pl.ds(e0, L)] + bp_v[pl.ds(e0, L)] + GLOBAL_MEAN
            for d in range(D):
                acc = acc + (plsc.load_gather(pbuf.at[buf], [rows, offs + d])
                             * plsc.load_gather(qbuf.at[buf], [rows, offp + d]))
            out_v[pl.ds(e0, L)] = acc
        inflight = nxt

    pltpu.sync_copy(out_v, out_hbm.at[pl.ds(base, b_per_w)])


@jax.jit
def kernel(SIDs, PIDs, P, Q, scientist_bias, paper_bias):
    B = SIDs.shape[0]
    b_per_w = B // NW
    sids = SIDs.astype(jnp.int32).reshape(NW, b_per_w)
    pids = PIDs.astype(jnp.int32).reshape(NW, b_per_w)
    p2 = P.reshape(-1, RW)
    q2 = Q.reshape(-1, RW)
    sb = scientist_bias.reshape(-1)
    pb = paper_bias.reshape(-1)

    mesh = plsc.VectorSubcoreMesh(core_axis_name="c", subcore_axis_name="s")
    f = pl.kernel(
        _mf_body,
        out_type=jax.ShapeDtypeStruct((B,), jnp.float32),
        mesh=mesh,
        compiler_params=pltpu.CompilerParams(needs_layout_passes=False),
        scratch_types=[
            pltpu.VMEM((b_per_w,), jnp.int32),
            pltpu.VMEM((b_per_w,), jnp.int32),
            pltpu.VMEM((b_per_w,), jnp.int32),
            pltpu.VMEM((b_per_w,), jnp.int32),
            pltpu.VMEM((2, CH, RW), jnp.float32),
            pltpu.VMEM((2, CH, RW), jnp.float32),
            pltpu.VMEM((b_per_w,), jnp.float32),
            pltpu.VMEM((b_per_w,), jnp.float32),
            pltpu.VMEM((b_per_w,), jnp.float32),
            pltpu.SemaphoreType.DMA,
            pltpu.SemaphoreType.DMA,
            pltpu.SemaphoreType.DMA,
        ],
    )
    return f(sids, pids, p2, q2, sb, pb)


# speedup vs baseline: 2.4190x; 2.4190x over previous
import jax
import jax.numpy as jnp
from jax import lax
from jax.experimental import pallas as pl
from jax.experimental.pallas import tpu as pltpu
from jax.experimental.pallas import tpu_sc as plsc

GLOBAL_MEAN = 3.5
D = 32
NC = 2
NS = 16
NW = NC * NS
L = 16
CE = 32
NCH = 16
GR = D * CE

N_P = 100000
N_Q = 1000000
NT_P = N_P // 128
NT_Q = N_Q // 128
PTH = NT_P * 128
QTH = NT_Q * 128
PTB = N_P - 160
QTB = N_Q - 128


def _granule_base(iv, th):
    ic = jnp.minimum(iv, th - 1)
    gi = lax.shift_left(lax.shift_right_logical(ic, 7), 6) + \
        jnp.bitwise_and(lax.shift_right_logical(ic, 4), 7)
    return gi, jnp.bitwise_and(ic, 15)


def _mf_body(sid_hbm, pid_hbm, pv_hbm, qv_hbm, pt_hbm, qt_hbm,
             sb_hbm, pb_hbm, out_hbm,
             sid_v, pid_v, pidx, qidx, ptidx, qtidx,
             pdst, qdst, ptd, qtd, bs_v, bp_v, out_v,
             semb, semp0, semp1, semq0, semq1):
    b_per_w = sid_v.shape[0]
    wid = lax.axis_index("s") * NC + lax.axis_index("c")
    base = wid * b_per_w
    semp = (semp0, semp1)
    semq = (semq0, semq1)
    lane = lax.iota(jnp.int32, L)

    pltpu.sync_copy(sid_hbm.at[pl.ds(base, b_per_w)], sid_v)
    pltpu.sync_copy(pid_hbm.at[pl.ds(base, b_per_w)], pid_v)

    bias_copies = []
    for k in range(b_per_w // 128):
        sl = pl.ds(k * 128, 128)
        bias_copies.append(pltpu.async_copy(sb_hbm.at[sid_v.at[sl]],
                                            bs_v.at[sl], semb))
        bias_copies.append(pltpu.async_copy(pb_hbm.at[pid_v.at[sl]],
                                            bp_v.at[sl], semb))

    def fire_one(c, buf, ids_v, view, tail, idx, tidx, dst, td, sem, th, tb,
                 nt):
        for sub in range(0, CE, L):
            iv = ids_v[pl.ds(c * CE + sub, L)]
            gi, _ = _granule_base(iv, th)
            fallback = sub + lane
            mi = 1 + lax.shift_right_arithmetic(iv - th, 31)
            tidx[pl.ds(buf * CE + sub, L)] = fallback + mi * (
                lax.shift_right_logical(iv - tb, 2) - fallback)
            for d in range(D):
                cd = (d >> 3) * nt * 64 + (d & 7) * 8
                idx[pl.ds(buf * GR + d * CE + sub, L)] = gi + cd
        for s in range(GR // 128):
            pltpu.async_copy(
                view.at[idx.at[pl.ds(buf * GR + s * 128, 128)]],
                dst.at[pl.ds(buf * GR + s * 128, 128)], sem[buf])
        pltpu.async_copy(tail.at[tidx.at[pl.ds(buf * CE, CE)]],
                         td.at[pl.ds(buf * CE, CE)], sem[buf])

    def fire(c, buf):
        fire_one(c, buf, pid_v, qv_hbm, qt_hbm, qidx, qtidx, qdst, qtd,
                 semq, QTH, QTB, NT_Q)
        fire_one(c, buf, sid_v, pv_hbm, pt_hbm, pidx, ptidx, pdst, ptd,
                 semp, PTH, PTB, NT_P)

    def drain(buf):
        pltpu.make_async_copy(qv_hbm.at[pl.ds(0, GR)],
                              qdst.at[pl.ds(buf * GR, GR)],
                              semq[buf]).wait()
        pltpu.make_async_copy(qt_hbm.at[pl.ds(0, CE)],
                              qtd.at[pl.ds(buf * CE, CE)],
                              semq[buf]).wait()
        pltpu.make_async_copy(pv_hbm.at[pl.ds(0, GR)],
                              pdst.at[pl.ds(buf * GR, GR)],
                              semp[buf]).wait()
        pltpu.make_async_copy(pt_hbm.at[pl.ds(0, CE)],
                              ptd.at[pl.ds(buf * CE, CE)],
                              semp[buf]).wait()

    def patch_tail(buf, ids_v, e0, sub, dst, td, th, tb):
        iv = ids_v[pl.ds(e0 + sub, L)]
        it = iv >= th
        ntail = plsc.all_reduce_population_count(it)

        @pl.when(ntail[0] > 0)
        def _():
            _, low = _granule_base(iv, th)
            off = lax.shift_left(jnp.bitwise_and(iv - tb, 3), 5)
            for d in range(D):
                rowv = buf * GR + d * CE + sub + lane
                tv = plsc.load_gather(
                    td, [buf * CE + sub + lane, off + d], mask=it)
                plsc.store_scatter(dst, [rowv, low], tv, mask=it)

    def compute(c, buf):
        for sub in range(0, CE, L):
            e0 = c * CE
            patch_tail(buf, sid_v, e0, sub, pdst, ptd, PTH, PTB)
            patch_tail(buf, pid_v, e0, sub, qdst, qtd, QTH, QTB)
            _, lows = _granule_base(sid_v[pl.ds(e0 + sub, L)], PTH)
            _, lowq = _granule_base(pid_v[pl.ds(e0 + sub, L)], QTH)
            sl = pl.ds(e0 + sub, L)
            acc = bs_v[sl] + bp_v[sl] + GLOBAL_MEAN
            for d in range(D):
                rowv = buf * GR + d * CE + sub + lane
                acc = acc + (plsc.load_gather(pdst, [rowv, lows])
                             * plsc.load_gather(qdst, [rowv, lowq]))
            out_v[sl] = acc

    fire(0, 0)
    for h in bias_copies:
        h.wait()

    def step(k, carry):
        c0 = 2 * k
        fire(c0 + 1, 1)
        drain(0)
        compute(c0, 0)

        @pl.when(c0 + 2 < NCH)
        def _():
            fire(c0 + 2, 0)

        drain(1)
        compute(c0 + 1, 1)
        return carry

    lax.fori_loop(0, NCH // 2, step, 0)
    pltpu.sync_copy(out_v, out_hbm.at[pl.ds(base, b_per_w)])


@jax.jit
def kernel(SIDs, PIDs, P, Q, scientist_bias, paper_bias):
    B = SIDs.shape[0]
    b_per_w = B // NW
    sids = SIDs.astype(jnp.int32)
    pids = PIDs.astype(jnp.int32)
    pv = (P[:PTH].T.reshape(4, 8, NT_P, 128).transpose(0, 2, 1, 3)
          .reshape(-1, 16))
    qv = (Q[:QTH].T.reshape(4, 8, NT_Q, 128).transpose(0, 2, 1, 3)
          .reshape(-1, 16))
    pt = P[PTB:].reshape(-1, 128)
    qt = Q[QTB:].reshape(-1, 128)
    sb = scientist_bias.reshape(-1)
    pb = paper_bias.reshape(-1)

    mesh = plsc.VectorSubcoreMesh(core_axis_name="c", subcore_axis_name="s")
    f = pl.kernel(
        _mf_body,
        out_type=jax.ShapeDtypeStruct((B,), jnp.float32),
        mesh=mesh,
        compiler_params=pltpu.CompilerParams(
            needs_layout_passes=False, use_tc_tiling_on_sc=False),
        scratch_types=[
            pltpu.VMEM((b_per_w,), jnp.int32),
            pltpu.VMEM((b_per_w,), jnp.int32),
            pltpu.VMEM((2 * GR,), jnp.int32),
            pltpu.VMEM((2 * GR,), jnp.int32),
            pltpu.VMEM((2 * CE,), jnp.int32),
            pltpu.VMEM((2 * CE,), jnp.int32),
            pltpu.VMEM((2 * GR, 16), jnp.float32),
            pltpu.VMEM((2 * GR, 16), jnp.float32),
            pltpu.VMEM((2 * CE, 128), jnp.float32),
            pltpu.VMEM((2 * CE, 128), jnp.float32),
            pltpu.VMEM((b_per_w,), jnp.float32),
            pltpu.VMEM((b_per_w,), jnp.float32),
            pltpu.VMEM((b_per_w,), jnp.float32),
            pltpu.SemaphoreType.DMA,
            pltpu.SemaphoreType.DMA,
            pltpu.SemaphoreType.DMA,
            pltpu.SemaphoreType.DMA,
            pltpu.SemaphoreType.DMA,
        ],
    )
    return f(sids, pids, pv, qv, pt, qt, sb, pb)
